# baseline (speedup 1.0000x reference)
import functools

import jax
import jax.numpy as jnp
from jax import lax
from jax.experimental import pallas as pl
from jax.experimental.pallas import tpu as pltpu
from jax.experimental.pallas import tpu_sc as plsc

NUM_EMBED = 1000000
EMBED_DIM = 64
BATCH = 16384
HIST = 50

_info = plsc.get_sparse_core_info()
NC, NS = _info.num_cores, _info.num_subcores
NW = NC * NS
CHUNK = 128
NBT = BATCH // (NW * CHUNK)
NCHUNK = HIST * NBT
NTILE = BATCH // CHUNK
D8 = EMBED_DIM // 8
BLK = 8 * CHUNK
NBUF = 4
DEPTH = 2
NG16 = EMBED_DIM // 16


def _make_kernel():
    mesh = plsc.VectorSubcoreMesh(core_axis_name="c", subcore_axis_name="s")

    @functools.partial(
        pl.kernel,
        mesh=mesh,
        out_type=jax.ShapeDtypeStruct((HIST, D8, NTILE, BLK), jnp.float32),
        compiler_params=pltpu.CompilerParams(
            use_tc_tiling_on_sc=False, needs_layout_passes=False
        ),
        scratch_types=[
            pltpu.VMEM((HIST, NBT, CHUNK), jnp.int32),
            pltpu.VMEM((NG16 * 2, 16), jnp.int32),
            [pltpu.VMEM((CHUNK, EMBED_DIM), jnp.float32) for _ in range(NBUF)],
            pltpu.VMEM((CHUNK, EMBED_DIM + 1), jnp.float32),
            [pltpu.VMEM((D8 * BLK,), jnp.float32) for _ in range(NBUF)],
            [pltpu.SemaphoreType.DMA for _ in range(NBUF)],
            [pltpu.SemaphoreType.DMA for _ in range(NBUF)],
        ],
    )
    def k(
        table_hbm, idx_hbm, out_hbm, idx_v, pvec_v, rows, r65, blks, gsems, psems
    ):
        wid = lax.axis_index("s") * NC + lax.axis_index("c")
        bt0 = wid * NBT
        pltpu.sync_copy(idx_hbm.at[wid], idx_v)

        dv = lax.iota(jnp.int32, 16)
        for c8 in range(8):
            pvec_v[c8] = dv + 16 * c8

        def fire_gather(g, p):
            h = g // NBT
            bt = g % NBT
            pltpu.async_copy(table_hbm.at[idx_v.at[h, bt]], rows[p], gsems[p])

        def drain_gather(p):
            pltpu.make_async_copy(
                table_hbm.at[pl.ds(0, CHUNK)], rows[p], gsems[p]
            ).wait()

        def permute(p):
            r = rows[p]
            b = blks[p]

            def rbody(t, carry):
                for j in range(8):
                    c = 8 * t + j
                    for kk in range(NG16):
                        r65[c, pl.ds(16 * kk, 16)] = r[c, pl.ds(16 * kk, 16)]
                return carry

            lax.fori_loop(0, CHUNK // 8, rbody, 0)

            def tbody(t, carry):
                for j in range(4):
                    d = 4 * t + j
                    base = ((d >> 3) << 10) + ((d & 7) << 7)
                    dsplat = jnp.full((16,), d, jnp.int32)
                    for c8 in range(8):
                        x = plsc.load_gather(r65, [pvec_v[c8], dsplat])
                        b[pl.ds(base + 16 * c8, 16)] = x
                return carry

            lax.fori_loop(0, EMBED_DIM // 4, tbody, 0)

        def fire_put(g, p):
            h = g // NBT
            bt = g % NBT
            for d8 in range(D8):
                pltpu.async_copy(
                    blks[p].at[pl.ds(d8 * BLK, BLK)],
                    out_hbm.at[h, d8, bt0 + bt],
                    psems[p],
                )

        def drain_put(p):
            for d8 in range(D8):
                pltpu.make_async_copy(
                    out_hbm.at[0, 0, 0],
                    blks[p].at[pl.ds(d8 * BLK, BLK)],
                    psems[p],
                ).wait()

        for j in range(DEPTH):
            fire_gather(j, j)

        def body(t, carry):
            for phase in range(NBUF):
                j = t * NBUF + phase
                s = phase
                sn = (phase + DEPTH) % NBUF
                jn = j + DEPTH

                @pl.when(jn < NCHUNK)
                def _():
                    @pl.when(jn >= NBUF)
                    def _():
                        drain_put(sn)

                    fire_gather(jn, sn)

                drain_gather(s)
                permute(s)
                fire_put(j, s)

            return carry

        lax.fori_loop(0, NCHUNK // NBUF, body, 0)
        for s in range(NBUF):
            drain_put(s)

    return k


_sc_gather = _make_kernel()


def kernel(inputs, vec_matrix):
    idx = (
        inputs.astype(jnp.int32)
        .reshape(NW, NBT, CHUNK, HIST)
        .transpose(0, 3, 1, 2)
    )
    raw = _sc_gather(vec_matrix, idx)
    out = (
        raw.reshape(HIST, D8, NTILE, 8, CHUNK)
        .transpose(2, 4, 0, 1, 3)
        .reshape(BATCH, HIST, EMBED_DIM)
    )
    return out


# speedup vs baseline: 3.0065x; 3.0065x over previous
import functools

import jax
import jax.numpy as jnp
from jax import lax
from jax.experimental import pallas as pl
from jax.experimental.pallas import tpu as pltpu
from jax.experimental.pallas import tpu_sc as plsc

NUM_EMBED = 1000000
EMBED_DIM = 64
BATCH = 16384
HIST = 50

_info = plsc.get_sparse_core_info()
NC, NS = _info.num_cores, _info.num_subcores
NW = NC * NS
CHUNK = 128
NBT = BATCH // (NW * CHUNK)
NCHUNK = HIST * NBT
NTILE = BATCH // CHUNK
D8 = EMBED_DIM // 8
BLK = 8 * CHUNK
NBUF = 4
DEPTH = 2
NG16 = EMBED_DIM // 16


def _make_kernel():
    mesh = plsc.VectorSubcoreMesh(core_axis_name="c", subcore_axis_name="s")

    @functools.partial(
        pl.kernel,
        mesh=mesh,
        out_type=jax.ShapeDtypeStruct((HIST, D8, NTILE, BLK), jnp.float32),
        compiler_params=pltpu.CompilerParams(
            use_tc_tiling_on_sc=False, needs_layout_passes=False
        ),
        scratch_types=[
            pltpu.VMEM((HIST, NBT, CHUNK), jnp.int32),
            pltpu.VMEM((NG16 * 2, 16), jnp.int32),
            [pltpu.VMEM((CHUNK, EMBED_DIM), jnp.float32) for _ in range(NBUF)],
            pltpu.VMEM((CHUNK, EMBED_DIM + 1), jnp.float32),
            [pltpu.VMEM((D8 * BLK,), jnp.float32) for _ in range(NBUF)],
            [pltpu.SemaphoreType.DMA for _ in range(NBUF)],
            [pltpu.SemaphoreType.DMA for _ in range(NBUF)],
        ],
    )
    def k(
        table_hbm, idx_hbm, out_hbm, idx_v, pvec_v, rows, r65, blks, gsems, psems
    ):
        wid = lax.axis_index("s") * NC + lax.axis_index("c")
        bt0 = wid * NBT
        pltpu.sync_copy(idx_hbm.at[wid], idx_v)

        dv = lax.iota(jnp.int32, 16)
        for c8 in range(8):
            pvec_v[c8] = dv + 16 * c8

        def fire_gather(g, p):
            h = g // NBT
            bt = g % NBT
            pltpu.async_copy(table_hbm.at[idx_v.at[h, bt]], rows[p], gsems[p])

        def drain_gather(p):
            pltpu.make_async_copy(
                table_hbm.at[pl.ds(0, CHUNK)], rows[p], gsems[p]
            ).wait()

        def permute(p):
            r = rows[p]
            b = blks[p]

            @functools.partial(plsc.parallel_loop, 0, CHUNK // 8, unroll=2)
            def rbody(t):
                for j in range(8):
                    c = 8 * t + j
                    for kk in range(NG16):
                        r65[c, pl.ds(16 * kk, 16)] = r[c, pl.ds(16 * kk, 16)]

            @functools.partial(
                plsc.parallel_loop, 0, EMBED_DIM // 4, unroll=2
            )
            def tbody(t):
                for j in range(4):
                    d = 4 * t + j
                    base = ((d >> 3) << 10) + ((d & 7) << 7)
                    dsplat = jnp.full((16,), d, jnp.int32)
                    for c8 in range(8):
                        x = plsc.load_gather(r65, [pvec_v[c8], dsplat])
                        b[pl.ds(base + 16 * c8, 16)] = x

        def fire_put(g, p):
            h = g // NBT
            bt = g % NBT
            for d8 in range(D8):
                pltpu.async_copy(
                    blks[p].at[pl.ds(d8 * BLK, BLK)],
                    out_hbm.at[h, d8, bt0 + bt],
                    psems[p],
                )

        def drain_put(p):
            for d8 in range(D8):
                pltpu.make_async_copy(
                    out_hbm.at[0, 0, 0],
                    blks[p].at[pl.ds(d8 * BLK, BLK)],
                    psems[p],
                ).wait()

        for j in range(DEPTH):
            fire_gather(j, j)

        def body(t, carry):
            for phase in range(NBUF):
                j = t * NBUF + phase
                s = phase
                sn = (phase + DEPTH) % NBUF
                jn = j + DEPTH

                @pl.when(jn < NCHUNK)
                def _():
                    @pl.when(jn >= NBUF)
                    def _():
                        drain_put(sn)

                    fire_gather(jn, sn)

                drain_gather(s)
                permute(s)
                fire_put(j, s)

            return carry

        lax.fori_loop(0, NCHUNK // NBUF, body, 0)
        for s in range(NBUF):
            drain_put(s)

    return k


_sc_gather = _make_kernel()


def kernel(inputs, vec_matrix):
    idx = (
        inputs.astype(jnp.int32)
        .reshape(NW, NBT, CHUNK, HIST)
        .transpose(0, 3, 1, 2)
    )
    raw = _sc_gather(vec_matrix, idx)
    out = (
        raw.reshape(HIST, D8, NTILE, 8, CHUNK)
        .transpose(2, 4, 0, 1, 3)
        .reshape(BATCH, HIST, EMBED_DIM)
    )
    return out
